# baseline (speedup 1.0000x reference)
import functools

import jax
import jax.numpy as jnp
from jax import lax
from jax.experimental import pallas as pl
from jax.experimental.pallas import tpu as pltpu
from jax.experimental.pallas import tpu_sc as plsc

N = 100000
E = 1600000
G = 64
D_IN, D_HID, D_OUT = 32, 16, 5

NC, NS = 2, 16
NW = NC * NS
TL = 256
EP = 1605632
ROWS = EP // TL
PAD_IDX = N
CH = 5000
NCH = N // CH
ZCH = 200
DH = D_HID // 2

_mesh = plsc.VectorSubcoreMesh(core_axis_name="c", subcore_axis_name="s",
                               num_cores=NC, num_subcores=NS)
_sc_params = pltpu.CompilerParams(use_tc_tiling_on_sc=False)
_sc_params_nl = pltpu.CompilerParams(use_tc_tiling_on_sc=False,
                                     needs_layout_passes=False)


DEG_RT = ROWS // NW
DEG_MB = 14
DEG_NMB = DEG_RT // DEG_MB


@functools.partial(
    pl.kernel,
    out_type=jax.ShapeDtypeStruct((NC, N, 8), jnp.float32),
    mesh=_mesh,
    compiler_params=_sc_params,
    scratch_types=[
        pltpu.VMEM((DEG_MB, TL), jnp.int32),
        pltpu.VMEM((TL, 8), jnp.float32),
        pltpu.VMEM((ZCH, 8), jnp.float32),
        pltpu.VMEM((2000, 8), jnp.float32),
        pltpu.VMEM_SHARED((N + 8, 8), jnp.float32),
        pltpu.SemaphoreType.DMA,
    ],
)
def _deg_kernel(dst_hbm, zeros_hbm, ones_hbm, out_hbm,
                didx, ones_v, zv, bb, acc, sem):
    c = lax.axis_index("c")
    s = lax.axis_index("s")
    pltpu.sync_copy(zeros_hbm, zv)
    pltpu.sync_copy(ones_hbm, ones_v)
    for j in range(2):
        k = s + j * NS

        @pl.when(k < NCH)
        def _():
            def zbody(m, carry):
                pltpu.sync_copy(zv, acc.at[pl.ds(k * CH + m * ZCH, ZCH)])
                return carry
            lax.fori_loop(0, CH // ZCH, zbody, 0)

    @pl.when(s == 0)
    def _():
        pltpu.sync_copy(zv.at[pl.ds(0, 8)], acc.at[pl.ds(N, 8)])

    plsc.subcore_barrier()
    base = (c * NS + s) * DEG_RT

    def body(i, carry):
        pltpu.sync_copy(dst_hbm.at[pl.ds(base + i * DEG_MB, DEG_MB)], didx)
        descs = [pltpu.async_copy(ones_v, acc.at[didx.at[j]], sem, add=True)
                 for j in range(DEG_MB)]
        for d in descs:
            d.wait()
        return carry

    lax.fori_loop(0, DEG_NMB, body, 0)
    plsc.subcore_barrier()
    for j in range(2):
        k = s + j * NS

        @pl.when(k < NCH)
        def _():
            for m in range(2):
                off = k * CH + m * 2000
                pltpu.sync_copy(acc.at[pl.ds(off, 2000)], bb)
                pltpu.sync_copy(bb, out_hbm.at[c, pl.ds(off, 2000)])
            off = k * CH + 4000
            pltpu.sync_copy(acc.at[pl.ds(off, 1000)], bb.at[pl.ds(0, 1000)])
            pltpu.sync_copy(bb.at[pl.ds(0, 1000)], out_hbm.at[c, pl.ds(off, 1000)])


HN = N // 2
PRT = ROWS // NW
PSUP = 14
FB = 1792
PCAP = EP // NW
TRASH = HN


@functools.partial(
    pl.kernel,
    out_type=[
        jax.ShapeDtypeStruct((NC, NW, PCAP), jnp.int32),
        jax.ShapeDtypeStruct((NC, NW, PCAP), jnp.int32),
        jax.ShapeDtypeStruct((NW, 16), jnp.int32),
    ],
    mesh=_mesh,
    compiler_params=_sc_params_nl,
    scratch_types=[
        pltpu.VMEM((PSUP, TL), jnp.int32),
        pltpu.VMEM((PSUP, TL), jnp.int32),
        pltpu.VMEM((FB + 32,), jnp.int32),
        pltpu.VMEM((FB + 32,), jnp.int32),
        pltpu.VMEM((FB + 32,), jnp.int32),
        pltpu.VMEM((FB + 32,), jnp.int32),
        pltpu.VMEM((16,), jnp.int32),
    ],
)
def _part_kernel(src_hbm, dst_hbm, psrc_hbm, pdst_hbm, pcnt_hbm,
                 eds, edd, bs0, bd0, bs1, bd1, cv):
    c = lax.axis_index("c")
    s = lax.axis_index("s")
    t = c * NS + s
    base = t * PRT
    iota16 = lax.broadcasted_iota(jnp.int32, (16,), 0)
    trash_d = jnp.full((16,), TRASH, jnp.int32)
    zero_s = jnp.zeros((16,), jnp.int32)

    def flush(buf_s, buf_d, h, f):
        pltpu.sync_copy(buf_s.at[pl.ds(0, FB)],
                        psrc_hbm.at[h, t, pl.ds(f * FB, FB)])
        pltpu.sync_copy(buf_d.at[pl.ds(0, FB)],
                        pdst_hbm.at[h, t, pl.ds(f * FB, FB)])

    def sup_body(i, carry):
        c0, c1, f0, f1 = carry
        pltpu.sync_copy(src_hbm.at[pl.ds(base + i * PSUP, PSUP)], eds)
        pltpu.sync_copy(dst_hbm.at[pl.ds(base + i * PSUP, PSUP)], edd)
        for j in range(PSUP):
            def vreg_body(k, cr):
                c0, c1, f0, f1 = cr
                sv = eds[j, pl.ds(k * 16, 16)]
                dv = edd[j, pl.ds(k * 16, 16)]
                hn_v = jnp.full((16,), HN, jnp.int32)
                one_v = jnp.full((16,), 1, jnp.int32)
                m0 = dv < hn_v
                cs0 = plsc.cumsum(m0.astype(jnp.int32))
                trash_pos = jnp.full((16,), FB + 16, jnp.int32)
                c0v = jnp.full((16,), c0, jnp.int32)
                c1v = jnp.full((16,), c1, jnp.int32)
                pos0 = jnp.where(m0, c0v + cs0 - one_v, trash_pos)
                pos1 = jnp.where(m0, trash_pos, c1v + iota16 - cs0)
                plsc.store_scatter(bs0, [pos0], sv)
                plsc.store_scatter(bd0, [pos0], dv)
                dl1 = dv - hn_v
                plsc.store_scatter(bs1, [pos1], sv)
                plsc.store_scatter(bd1, [pos1], dl1)
                n0 = jnp.max(cs0)
                c0 = c0 + n0
                c1 = c1 + (16 - n0)
                cond0 = c0 >= FB
                cond1 = c1 >= FB

                @pl.when(cond0)
                def _():
                    flush(bs0, bd0, 0, f0)
                    bs0[pl.ds(0, 16)] = bs0[pl.ds(FB, 16)]
                    bd0[pl.ds(0, 16)] = bd0[pl.ds(FB, 16)]

                @pl.when(cond1)
                def _():
                    flush(bs1, bd1, 1, f1)
                    bs1[pl.ds(0, 16)] = bs1[pl.ds(FB, 16)]
                    bd1[pl.ds(0, 16)] = bd1[pl.ds(FB, 16)]

                c0 = jnp.where(cond0, c0 - FB, c0)
                f0 = jnp.where(cond0, f0 + 1, f0)
                c1 = jnp.where(cond1, c1 - FB, c1)
                f1 = jnp.where(cond1, f1 + 1, f1)
                return (c0, c1, f0, f1)

            c0, c1, f0, f1 = lax.fori_loop(0, TL // 16, vreg_body,
                                           (c0, c1, f0, f1))
        return (c0, c1, f0, f1)

    z = jnp.int32(0)
    c0, c1, f0, f1 = lax.fori_loop(0, PRT // PSUP, sup_body, (z, z, z, z))

    @pl.when(c0 > 0)
    def _():
        def pbody(i, carry):
            @pl.when(i * 16 < FB - c0)
            def _():
                bs0[pl.ds(c0 + i * 16, 16)] = zero_s
                bd0[pl.ds(c0 + i * 16, 16)] = trash_d
            return carry
        lax.fori_loop(0, FB // 16, pbody, 0)
        flush(bs0, bd0, 0, f0)

    @pl.when(c1 > 0)
    def _():
        def pbody(i, carry):
            @pl.when(i * 16 < FB - c1)
            def _():
                bs1[pl.ds(c1 + i * 16, 16)] = zero_s
                bd1[pl.ds(c1 + i * 16, 16)] = trash_d
            return carry
        lax.fori_loop(0, FB // 16, pbody, 0)
        flush(bs1, bd1, 1, f1)

    nsup0 = f0 + (c0 > 0).astype(jnp.int32)
    nsup1 = f1 + (c1 > 0).astype(jnp.int32)
    zv16 = jnp.zeros((16,), jnp.int32)
    cv[...] = jnp.where(iota16 == zv16, jnp.full((16,), nsup0, jnp.int32),
                        jnp.where(iota16 == jnp.full((16,), 1, jnp.int32),
                                  jnp.full((16,), nsup1, jnp.int32), zv16))
    pltpu.sync_copy(cv, pcnt_hbm.at[t])


NCH_H = HN // CH


@functools.partial(
    pl.kernel,
    out_type=jax.ShapeDtypeStruct((N, D_HID), jnp.float32),
    mesh=_mesh,
    compiler_params=_sc_params_nl,
    scratch_types=[
        pltpu.VMEM((FB,), jnp.int32),
        pltpu.VMEM((7, TL), jnp.int32),
        pltpu.VMEM((FB, D_HID), jnp.float32),
        pltpu.VMEM((ZCH, D_HID), jnp.float32),
        pltpu.VMEM((16,), jnp.int32),
        pltpu.VMEM_SHARED((HN + 8, D_HID), jnp.float32),
        pltpu.SemaphoreType.DMA,
        pltpu.SemaphoreType.DMA,
    ],
)
def _agg_kernel(psrc_hbm, pdst_hbm, pcnt_hbm, y_hbm, zeros_hbm, out_hbm,
                sidx, didx, rows_v, zb, cv, acc, gsem, ssem):
    c = lax.axis_index("c")
    s = lax.axis_index("s")
    iota16 = lax.broadcasted_iota(jnp.int32, (16,), 0)
    pltpu.sync_copy(zeros_hbm, zb)
    for j in range(2):
        k = s + j * NS

        @pl.when(k < NCH_H)
        def _():
            def zbody(m, carry):
                pltpu.sync_copy(zb, acc.at[pl.ds(k * CH + m * ZCH, ZCH)])
                return carry
            lax.fori_loop(0, CH // ZCH, zbody, 0)

    @pl.when(s == 0)
    def _():
        pltpu.sync_copy(zb.at[pl.ds(0, 8)], acc.at[pl.ds(HN, 8)])

    plsc.subcore_barrier()

    for pi in range(2):
        p = 2 * s + pi
        pltpu.sync_copy(pcnt_hbm.at[p], cv)
        nsup = jnp.max(jnp.where(iota16 == jnp.full((16,), c, jnp.int32),
                                 cv[...], jnp.zeros((16,), jnp.int32)))

        def sup_body(i, carry):
            off = i * FB
            pltpu.sync_copy(psrc_hbm.at[c, p, pl.ds(off, FB)], sidx)
            for j in range(7):
                pltpu.sync_copy(pdst_hbm.at[c, p, pl.ds(off + j * TL, TL)],
                                didx.at[j])
            gd = [pltpu.async_copy(y_hbm.at[sidx.at[pl.ds(j * TL, TL)]],
                                   rows_v.at[pl.ds(j * TL, TL)], gsem)
                  for j in range(7)]
            sd = []
            for j in range(7):
                gd[j].wait()
                sd.append(pltpu.async_copy(rows_v.at[pl.ds(j * TL, TL)],
                                           acc.at[didx.at[j]], ssem, add=True))
            for d in sd:
                d.wait()
            return carry

        lax.fori_loop(0, nsup, sup_body, 0)

    plsc.subcore_barrier()
    for j in range(2):
        k = s + j * NS

        @pl.when(k < NCH_H)
        def _():
            def obody(m, carry):
                off = k * CH + m * 1000
                pltpu.sync_copy(acc.at[pl.ds(off, 1000)],
                                rows_v.at[pl.ds(0, 1000)])
                pltpu.sync_copy(rows_v.at[pl.ds(0, 1000)],
                                out_hbm.at[pl.ds(c * HN + off, 1000)])
                return carry
            lax.fori_loop(0, CH // 1000, obody, 0)


BN = 5000
NB = N // BN


def _mm1_body(x_ref, d0_ref, d1_ref, w_ref, y_ref, dv_ref):
    deg = d0_ref[..., 0:1] + d1_ref[..., 0:1] + 1.0
    dinv = lax.rsqrt(deg)
    y_ref[...] = dinv * jnp.dot(x_ref[...], w_ref[...],
                                preferred_element_type=jnp.float32)
    dv_ref[...] = dinv


def _mm1_call(x, d0, d1, W1):
    return pl.pallas_call(
        _mm1_body,
        grid=(NB,),
        in_specs=[
            pl.BlockSpec((BN, D_IN), lambda i: (i, 0)),
            pl.BlockSpec((BN, 8), lambda i: (i, 0)),
            pl.BlockSpec((BN, 8), lambda i: (i, 0)),
            pl.BlockSpec((D_IN, D_HID), lambda i: (0, 0)),
        ],
        out_specs=[
            pl.BlockSpec((BN, D_HID), lambda i: (i, 0)),
            pl.BlockSpec((BN, 1), lambda i: (i, 0)),
        ],
        out_shape=[
            jax.ShapeDtypeStruct((N, D_HID), jnp.float32),
            jax.ShapeDtypeStruct((N, 1), jnp.float32),
        ],
    )(x, d0, d1, W1)


def _mid_body(a_ref, y1_ref, dv_ref, b1_ref, w2_ref, y2_ref):
    dinv = dv_ref[...]
    h = dinv * (a_ref[...] + y1_ref[...]) + b1_ref[...]
    h = jnp.maximum(h, 0.0)
    y2_ref[...] = dinv * jnp.dot(h, w2_ref[...],
                                 preferred_element_type=jnp.float32)


def _mid_call(a, y1, dv, b1, W2):
    return pl.pallas_call(
        _mid_body,
        grid=(NB,),
        in_specs=[
            pl.BlockSpec((BN, D_HID), lambda i: (i, 0)),
            pl.BlockSpec((BN, D_HID), lambda i: (i, 0)),
            pl.BlockSpec((BN, 1), lambda i: (i, 0)),
            pl.BlockSpec((1, D_HID), lambda i: (0, 0)),
            pl.BlockSpec((D_HID, D_HID), lambda i: (0, 0)),
        ],
        out_specs=pl.BlockSpec((BN, D_HID), lambda i: (i, 0)),
        out_shape=jax.ShapeDtypeStruct((N, D_HID), jnp.float32),
    )(a, y1, dv, b1, W2)


def _fin_body(a_ref, y2_ref, dv_ref, b2_ref, bt_ref, wl_ref, bl_ref,
              out_ref, sums, cnt):
    i = pl.program_id(0)

    @pl.when(i == 0)
    def _():
        sums[...] = jnp.zeros_like(sums)
        cnt[...] = jnp.zeros_like(cnt)

    h2 = dv_ref[...] * (a_ref[...] + y2_ref[...]) + b2_ref[...]
    oh = (bt_ref[...] == lax.broadcasted_iota(jnp.int32, (BN, G), 1))
    oh = oh.astype(jnp.float32)
    sums[...] += lax.dot_general(oh, h2, (((0,), (0,)), ((), ())),
                                 preferred_element_type=jnp.float32)
    cnt[...] += lax.dot_general(oh, jnp.ones((BN, 1), jnp.float32),
                                (((0,), (0,)), ((), ())),
                                preferred_element_type=jnp.float32)

    @pl.when(i == NB - 1)
    def _():
        pooled = sums[...] / jnp.maximum(cnt[...], 1.0)
        logits = jnp.dot(pooled, wl_ref[...],
                         preferred_element_type=jnp.float32) + bl_ref[...]
        m = jnp.max(logits, axis=1, keepdims=True)
        e = jnp.exp(logits - m)
        out_ref[...] = e / jnp.sum(e, axis=1, keepdims=True)


def _fin_call(a, y2, dv, b2, bt, Wl, bl):
    return pl.pallas_call(
        _fin_body,
        grid=(NB,),
        in_specs=[
            pl.BlockSpec((BN, D_HID), lambda i: (i, 0)),
            pl.BlockSpec((BN, D_HID), lambda i: (i, 0)),
            pl.BlockSpec((BN, 1), lambda i: (i, 0)),
            pl.BlockSpec((1, D_HID), lambda i: (0, 0)),
            pl.BlockSpec((BN, 1), lambda i: (i, 0)),
            pl.BlockSpec((D_HID, D_OUT), lambda i: (0, 0)),
            pl.BlockSpec((1, D_OUT), lambda i: (0, 0)),
        ],
        out_specs=pl.BlockSpec((G, D_OUT), lambda i: (0, 0)),
        out_shape=jax.ShapeDtypeStruct((G, D_OUT), jnp.float32),
        scratch_shapes=[
            pltpu.VMEM((G, D_HID), jnp.float32),
            pltpu.VMEM((G, 1), jnp.float32),
        ],
    )(a, y2, dv, b2, bt, Wl, bl)


def kernel(x, edge_index, batch, W1, b1, W2, b2, Wl, bl):
    ei = edge_index.astype(jnp.int32)
    src = ei[0]
    dst = ei[1]
    pad = EP - E
    srcp = jnp.concatenate([src, jnp.zeros((pad,), jnp.int32)]).reshape(ROWS, TL)
    dstp = jnp.concatenate([dst, jnp.full((pad,), PAD_IDX, jnp.int32)]
                           ).reshape(ROWS, TL)
    zeros8_c = jnp.zeros((ZCH, 8), jnp.float32)
    zeros16_c = jnp.zeros((ZCH, D_HID), jnp.float32)
    ones_c = jnp.ones((TL, 8), jnp.float32)

    deg2 = _deg_kernel(dstp, zeros8_c, ones_c)
    psrc, pdst, pcnt = _part_kernel(srcp, dstp)
    y1, dinv = _mm1_call(x, deg2[0], deg2[1], W1)
    agg1 = _agg_kernel(psrc, pdst, pcnt, y1, zeros16_c)
    y2 = _mid_call(agg1, y1, dinv,
                   b1.reshape(1, D_HID), W2)
    agg2 = _agg_kernel(psrc, pdst, pcnt, y2, zeros16_c)
    return _fin_call(agg2, y2, dinv,
                     b2.reshape(1, D_HID), batch.astype(jnp.int32).reshape(N, 1),
                     Wl, bl.reshape(1, D_OUT))


# speedup vs baseline: 1.0753x; 1.0753x over previous
import functools

import jax
import jax.numpy as jnp
from jax import lax
from jax.experimental import pallas as pl
from jax.experimental.pallas import tpu as pltpu
from jax.experimental.pallas import tpu_sc as plsc

N = 100000
E = 1600000
G = 64
D_IN, D_HID, D_OUT = 32, 16, 5

NC, NS = 2, 16
NW = NC * NS
TL = 256
EP = 1605632
ROWS = EP // TL
PAD_IDX = N
CH = 5000
NCH = N // CH
ZCH = 200
DH = D_HID // 2

_mesh = plsc.VectorSubcoreMesh(core_axis_name="c", subcore_axis_name="s",
                               num_cores=NC, num_subcores=NS)
_sc_params = pltpu.CompilerParams(use_tc_tiling_on_sc=False)
_sc_params_nl = pltpu.CompilerParams(use_tc_tiling_on_sc=False,
                                     needs_layout_passes=False)


DEG_RT = ROWS // NW
DEG_MB = 14
DEG_NMB = DEG_RT // DEG_MB


@functools.partial(
    pl.kernel,
    out_type=jax.ShapeDtypeStruct((NC, N, 8), jnp.float32),
    mesh=_mesh,
    compiler_params=_sc_params,
    scratch_types=[
        pltpu.VMEM((DEG_MB, TL), jnp.int32),
        pltpu.VMEM((TL, 8), jnp.float32),
        pltpu.VMEM((ZCH, 8), jnp.float32),
        pltpu.VMEM((2000, 8), jnp.float32),
        pltpu.VMEM_SHARED((N + 8, 8), jnp.float32),
        pltpu.SemaphoreType.DMA,
    ],
)
def _deg_kernel(dst_hbm, zeros_hbm, ones_hbm, out_hbm,
                didx, ones_v, zv, bb, acc, sem):
    c = lax.axis_index("c")
    s = lax.axis_index("s")
    pltpu.sync_copy(zeros_hbm, zv)
    pltpu.sync_copy(ones_hbm, ones_v)
    for j in range(2):
        k = s + j * NS

        @pl.when(k < NCH)
        def _():
            def zbody(m, carry):
                pltpu.sync_copy(zv, acc.at[pl.ds(k * CH + m * ZCH, ZCH)])
                return carry
            lax.fori_loop(0, CH // ZCH, zbody, 0)

    @pl.when(s == 0)
    def _():
        pltpu.sync_copy(zv.at[pl.ds(0, 8)], acc.at[pl.ds(N, 8)])

    plsc.subcore_barrier()
    base = (c * NS + s) * DEG_RT

    def body(i, carry):
        pltpu.sync_copy(dst_hbm.at[pl.ds(base + i * DEG_MB, DEG_MB)], didx)
        descs = [pltpu.async_copy(ones_v, acc.at[didx.at[j]], sem, add=True)
                 for j in range(DEG_MB)]
        for d in descs:
            d.wait()
        return carry

    lax.fori_loop(0, DEG_NMB, body, 0)
    plsc.subcore_barrier()
    for j in range(2):
        k = s + j * NS

        @pl.when(k < NCH)
        def _():
            for m in range(2):
                off = k * CH + m * 2000
                pltpu.sync_copy(acc.at[pl.ds(off, 2000)], bb)
                pltpu.sync_copy(bb, out_hbm.at[c, pl.ds(off, 2000)])
            off = k * CH + 4000
            pltpu.sync_copy(acc.at[pl.ds(off, 1000)], bb.at[pl.ds(0, 1000)])
            pltpu.sync_copy(bb.at[pl.ds(0, 1000)], out_hbm.at[c, pl.ds(off, 1000)])


HN = N // 2
PRT = ROWS // NW
PSUP = 14
FB = 1792
PCAP = EP // NW
TRASH = HN


@functools.partial(
    pl.kernel,
    out_type=[
        jax.ShapeDtypeStruct((NC, NW, PCAP), jnp.int32),
        jax.ShapeDtypeStruct((NC, NW, PRT, TL), jnp.int32),
        jax.ShapeDtypeStruct((NW, 16), jnp.int32),
    ],
    mesh=_mesh,
    compiler_params=_sc_params_nl,
    scratch_types=[
        pltpu.VMEM((PSUP, TL), jnp.int32),
        pltpu.VMEM((PSUP, TL), jnp.int32),
        pltpu.VMEM((FB + 32,), jnp.int32),
        pltpu.VMEM((8, TL), jnp.int32),
        pltpu.VMEM((FB + 32,), jnp.int32),
        pltpu.VMEM((8, TL), jnp.int32),
        pltpu.VMEM((16,), jnp.int32),
    ],
)
def _part_kernel(src_hbm, dst_hbm, psrc_hbm, pdst_hbm, pcnt_hbm,
                 eds, edd, bs0, bd0, bs1, bd1, cv):
    c = lax.axis_index("c")
    s = lax.axis_index("s")
    t = c * NS + s
    base = t * PRT
    iota16 = lax.broadcasted_iota(jnp.int32, (16,), 0)
    trash_d = jnp.full((16,), TRASH, jnp.int32)
    zero_s = jnp.zeros((16,), jnp.int32)

    def flush(buf_s, buf_d, h, f):
        pltpu.sync_copy(buf_s.at[pl.ds(0, FB)],
                        psrc_hbm.at[h, t, pl.ds(f * FB, FB)])
        pltpu.sync_copy(buf_d.at[pl.ds(0, 7)],
                        pdst_hbm.at[h, t, pl.ds(f * 7, 7)])

    def sup_body(i, carry):
        c0, c1, f0, f1 = carry
        pltpu.sync_copy(src_hbm.at[pl.ds(base + i * PSUP, PSUP)], eds)
        pltpu.sync_copy(dst_hbm.at[pl.ds(base + i * PSUP, PSUP)], edd)
        for j in range(PSUP):
            def vreg_body(k, cr):
                c0, c1, f0, f1 = cr
                sv = eds[j, pl.ds(k * 16, 16)]
                dv = edd[j, pl.ds(k * 16, 16)]
                hn_v = jnp.full((16,), HN, jnp.int32)
                one_v = jnp.full((16,), 1, jnp.int32)
                m0 = dv < hn_v
                cs0 = plsc.cumsum(m0.astype(jnp.int32))
                trash_pos = jnp.full((16,), FB + 16, jnp.int32)
                c0v = jnp.full((16,), c0, jnp.int32)
                c1v = jnp.full((16,), c1, jnp.int32)
                pos0 = jnp.where(m0, c0v + cs0 - one_v, trash_pos)
                pos1 = jnp.where(m0, trash_pos, c1v + iota16 - cs0)
                plsc.store_scatter(bs0, [pos0], sv)
                plsc.store_scatter(bd0, [pos0 >> 8, pos0 & 255], dv)
                dl1 = dv - hn_v
                plsc.store_scatter(bs1, [pos1], sv)
                plsc.store_scatter(bd1, [pos1 >> 8, pos1 & 255], dl1)
                n0 = jnp.max(cs0)
                c0 = c0 + n0
                c1 = c1 + (16 - n0)
                cond0 = c0 >= FB
                cond1 = c1 >= FB

                @pl.when(cond0)
                def _():
                    flush(bs0, bd0, 0, f0)
                    bs0[pl.ds(0, 16)] = bs0[pl.ds(FB, 16)]
                    bd0[0, pl.ds(0, 16)] = bd0[7, pl.ds(0, 16)]

                @pl.when(cond1)
                def _():
                    flush(bs1, bd1, 1, f1)
                    bs1[pl.ds(0, 16)] = bs1[pl.ds(FB, 16)]
                    bd1[0, pl.ds(0, 16)] = bd1[7, pl.ds(0, 16)]

                c0 = jnp.where(cond0, c0 - FB, c0)
                f0 = jnp.where(cond0, f0 + 1, f0)
                c1 = jnp.where(cond1, c1 - FB, c1)
                f1 = jnp.where(cond1, f1 + 1, f1)
                return (c0, c1, f0, f1)

            c0, c1, f0, f1 = lax.fori_loop(0, TL // 16, vreg_body,
                                           (c0, c1, f0, f1))
        return (c0, c1, f0, f1)

    z = jnp.int32(0)
    c0, c1, f0, f1 = lax.fori_loop(0, PRT // PSUP, sup_body, (z, z, z, z))

    @pl.when(c0 > 0)
    def _():
        def pbody(i, carry):
            @pl.when(i * 16 < FB - c0)
            def _():
                posv = jnp.full((16,), c0 + i * 16, jnp.int32) + iota16
                bs0[pl.ds(c0 + i * 16, 16)] = zero_s
                plsc.store_scatter(bd0, [posv >> 8, posv & 255], trash_d)
            return carry
        lax.fori_loop(0, FB // 16, pbody, 0)
        flush(bs0, bd0, 0, f0)

    @pl.when(c1 > 0)
    def _():
        def pbody(i, carry):
            @pl.when(i * 16 < FB - c1)
            def _():
                posv = jnp.full((16,), c1 + i * 16, jnp.int32) + iota16
                bs1[pl.ds(c1 + i * 16, 16)] = zero_s
                plsc.store_scatter(bd1, [posv >> 8, posv & 255], trash_d)
            return carry
        lax.fori_loop(0, FB // 16, pbody, 0)
        flush(bs1, bd1, 1, f1)

    nsup0 = f0 + (c0 > 0).astype(jnp.int32)
    nsup1 = f1 + (c1 > 0).astype(jnp.int32)
    zv16 = jnp.zeros((16,), jnp.int32)
    cv[...] = jnp.where(iota16 == zv16, jnp.full((16,), nsup0, jnp.int32),
                        jnp.where(iota16 == jnp.full((16,), 1, jnp.int32),
                                  jnp.full((16,), nsup1, jnp.int32), zv16))
    pltpu.sync_copy(cv, pcnt_hbm.at[t])


NCH_H = HN // CH


@functools.partial(
    pl.kernel,
    out_type=jax.ShapeDtypeStruct((N, D_HID), jnp.float32),
    mesh=_mesh,
    compiler_params=_sc_params_nl,
    scratch_types=[
        pltpu.VMEM((FB,), jnp.int32),
        pltpu.VMEM((7, TL), jnp.int32),
        pltpu.VMEM((FB, D_HID), jnp.float32),
        pltpu.VMEM((ZCH, D_HID), jnp.float32),
        pltpu.VMEM((16,), jnp.int32),
        pltpu.VMEM_SHARED((HN + 8, D_HID), jnp.float32),
        pltpu.SemaphoreType.DMA,
        pltpu.SemaphoreType.DMA,
    ],
)
def _agg_kernel(psrc_hbm, pdst_hbm, pcnt_hbm, y_hbm, zeros_hbm, out_hbm,
                sidx, didx, rows_v, zb, cv, acc, gsem, ssem):
    c = lax.axis_index("c")
    s = lax.axis_index("s")
    iota16 = lax.broadcasted_iota(jnp.int32, (16,), 0)
    pltpu.sync_copy(zeros_hbm, zb)
    for j in range(2):
        k = s + j * NS

        @pl.when(k < NCH_H)
        def _():
            def zbody(m, carry):
                pltpu.sync_copy(zb, acc.at[pl.ds(k * CH + m * ZCH, ZCH)])
                return carry
            lax.fori_loop(0, CH // ZCH, zbody, 0)

    @pl.when(s == 0)
    def _():
        pltpu.sync_copy(zb.at[pl.ds(0, 8)], acc.at[pl.ds(HN, 8)])

    plsc.subcore_barrier()

    for pi in range(2):
        p = 2 * s + pi
        pltpu.sync_copy(pcnt_hbm.at[p], cv)
        nsup = jnp.max(jnp.where(iota16 == jnp.full((16,), c, jnp.int32),
                                 cv[...], jnp.zeros((16,), jnp.int32)))

        def sup_body(i, carry):
            off = i * FB
            pltpu.sync_copy(psrc_hbm.at[c, p, pl.ds(off, FB)], sidx)
            pltpu.sync_copy(pdst_hbm.at[c, p, pl.ds(i * 7, 7)], didx)
            gd = [pltpu.async_copy(y_hbm.at[sidx.at[pl.ds(j * TL, TL)]],
                                   rows_v.at[pl.ds(j * TL, TL)], gsem)
                  for j in range(7)]
            sd = []
            for j in range(7):
                gd[j].wait()
                sd.append(pltpu.async_copy(rows_v.at[pl.ds(j * TL, TL)],
                                           acc.at[didx.at[j]], ssem, add=True))
            for d in sd:
                d.wait()
            return carry

        lax.fori_loop(0, nsup, sup_body, 0)

    plsc.subcore_barrier()
    for j in range(2):
        k = s + j * NS

        @pl.when(k < NCH_H)
        def _():
            def obody(m, carry):
                off = k * CH + m * 1000
                pltpu.sync_copy(acc.at[pl.ds(off, 1000)],
                                rows_v.at[pl.ds(0, 1000)])
                pltpu.sync_copy(rows_v.at[pl.ds(0, 1000)],
                                out_hbm.at[pl.ds(c * HN + off, 1000)])
                return carry
            lax.fori_loop(0, CH // 1000, obody, 0)


BN = 5000
NB = N // BN


def _mm1_body(x_ref, d0_ref, d1_ref, w_ref, y_ref, dv_ref):
    deg = d0_ref[..., 0:1] + d1_ref[..., 0:1] + 1.0
    dinv = lax.rsqrt(deg)
    y_ref[...] = dinv * jnp.dot(x_ref[...], w_ref[...],
                                preferred_element_type=jnp.float32)
    dv_ref[...] = dinv


def _mm1_call(x, d0, d1, W1):
    return pl.pallas_call(
        _mm1_body,
        grid=(NB,),
        in_specs=[
            pl.BlockSpec((BN, D_IN), lambda i: (i, 0)),
            pl.BlockSpec((BN, 8), lambda i: (i, 0)),
            pl.BlockSpec((BN, 8), lambda i: (i, 0)),
            pl.BlockSpec((D_IN, D_HID), lambda i: (0, 0)),
        ],
        out_specs=[
            pl.BlockSpec((BN, D_HID), lambda i: (i, 0)),
            pl.BlockSpec((BN, 1), lambda i: (i, 0)),
        ],
        out_shape=[
            jax.ShapeDtypeStruct((N, D_HID), jnp.float32),
            jax.ShapeDtypeStruct((N, 1), jnp.float32),
        ],
    )(x, d0, d1, W1)


def _mid_body(a_ref, y1_ref, dv_ref, b1_ref, w2_ref, y2_ref):
    dinv = dv_ref[...]
    h = dinv * (a_ref[...] + y1_ref[...]) + b1_ref[...]
    h = jnp.maximum(h, 0.0)
    y2_ref[...] = dinv * jnp.dot(h, w2_ref[...],
                                 preferred_element_type=jnp.float32)


def _mid_call(a, y1, dv, b1, W2):
    return pl.pallas_call(
        _mid_body,
        grid=(NB,),
        in_specs=[
            pl.BlockSpec((BN, D_HID), lambda i: (i, 0)),
            pl.BlockSpec((BN, D_HID), lambda i: (i, 0)),
            pl.BlockSpec((BN, 1), lambda i: (i, 0)),
            pl.BlockSpec((1, D_HID), lambda i: (0, 0)),
            pl.BlockSpec((D_HID, D_HID), lambda i: (0, 0)),
        ],
        out_specs=pl.BlockSpec((BN, D_HID), lambda i: (i, 0)),
        out_shape=jax.ShapeDtypeStruct((N, D_HID), jnp.float32),
    )(a, y1, dv, b1, W2)


def _fin_body(a_ref, y2_ref, dv_ref, b2_ref, bt_ref, wl_ref, bl_ref,
              out_ref, sums, cnt):
    i = pl.program_id(0)

    @pl.when(i == 0)
    def _():
        sums[...] = jnp.zeros_like(sums)
        cnt[...] = jnp.zeros_like(cnt)

    h2 = dv_ref[...] * (a_ref[...] + y2_ref[...]) + b2_ref[...]
    oh = (bt_ref[...] == lax.broadcasted_iota(jnp.int32, (BN, G), 1))
    oh = oh.astype(jnp.float32)
    sums[...] += lax.dot_general(oh, h2, (((0,), (0,)), ((), ())),
                                 preferred_element_type=jnp.float32)
    cnt[...] += lax.dot_general(oh, jnp.ones((BN, 1), jnp.float32),
                                (((0,), (0,)), ((), ())),
                                preferred_element_type=jnp.float32)

    @pl.when(i == NB - 1)
    def _():
        pooled = sums[...] / jnp.maximum(cnt[...], 1.0)
        logits = jnp.dot(pooled, wl_ref[...],
                         preferred_element_type=jnp.float32) + bl_ref[...]
        m = jnp.max(logits, axis=1, keepdims=True)
        e = jnp.exp(logits - m)
        out_ref[...] = e / jnp.sum(e, axis=1, keepdims=True)


def _fin_call(a, y2, dv, b2, bt, Wl, bl):
    return pl.pallas_call(
        _fin_body,
        grid=(NB,),
        in_specs=[
            pl.BlockSpec((BN, D_HID), lambda i: (i, 0)),
            pl.BlockSpec((BN, D_HID), lambda i: (i, 0)),
            pl.BlockSpec((BN, 1), lambda i: (i, 0)),
            pl.BlockSpec((1, D_HID), lambda i: (0, 0)),
            pl.BlockSpec((BN, 1), lambda i: (i, 0)),
            pl.BlockSpec((D_HID, D_OUT), lambda i: (0, 0)),
            pl.BlockSpec((1, D_OUT), lambda i: (0, 0)),
        ],
        out_specs=pl.BlockSpec((G, D_OUT), lambda i: (0, 0)),
        out_shape=jax.ShapeDtypeStruct((G, D_OUT), jnp.float32),
        scratch_shapes=[
            pltpu.VMEM((G, D_HID), jnp.float32),
            pltpu.VMEM((G, 1), jnp.float32),
        ],
    )(a, y2, dv, b2, bt, Wl, bl)


def kernel(x, edge_index, batch, W1, b1, W2, b2, Wl, bl):
    ei = edge_index.astype(jnp.int32)
    src = ei[0]
    dst = ei[1]
    pad = EP - E
    srcp = jnp.concatenate([src, jnp.zeros((pad,), jnp.int32)]).reshape(ROWS, TL)
    dstp = jnp.concatenate([dst, jnp.full((pad,), PAD_IDX, jnp.int32)]
                           ).reshape(ROWS, TL)
    zeros8_c = jnp.zeros((ZCH, 8), jnp.float32)
    zeros16_c = jnp.zeros((ZCH, D_HID), jnp.float32)
    ones_c = jnp.ones((TL, 8), jnp.float32)

    deg2 = _deg_kernel(dstp, zeros8_c, ones_c)
    psrc, pdst, pcnt = _part_kernel(srcp, dstp)
    y1, dinv = _mm1_call(x, deg2[0], deg2[1], W1)
    agg1 = _agg_kernel(psrc, pdst, pcnt, y1, zeros16_c)
    y2 = _mid_call(agg1, y1, dinv,
                   b1.reshape(1, D_HID), W2)
    agg2 = _agg_kernel(psrc, pdst, pcnt, y2, zeros16_c)
    return _fin_call(agg2, y2, dinv,
                     b2.reshape(1, D_HID), batch.astype(jnp.int32).reshape(N, 1),
                     Wl, bl.reshape(1, D_OUT))
